# baseline (speedup 1.0000x reference)
import functools

import jax
import jax.numpy as jnp
from jax import lax
from jax.experimental import pallas as pl
from jax.experimental.pallas import tpu as pltpu
from jax.experimental.pallas import tpu_sc as plsc

N = 10000
E = 320000
D = 128
HALF = 64
CW = 8
K = 128
NS = 16
G = 16
NSUP = 10
NCHUNK = NSUP * G
EP = NCHUNK * NS * K
NCHUNK_TOT = EP // K
RPT = N // NS
ACCR = N + NS
RPTZ = ACCR // NS


def _make_agg(with_cnt, nb, gs):
    mesh = plsc.VectorSubcoreMesh(core_axis_name="c", subcore_axis_name="s")

    out_type = [jax.ShapeDtypeStruct((N, D), jnp.float32)]
    scratch = [
        pltpu.VMEM_SHARED((ACCR, HALF), jnp.float32),
        pltpu.VMEM_SHARED((N, HALF), jnp.float32),
        pltpu.VMEM((gs, 2, K), jnp.int32),
        [pltpu.VMEM((K, HALF), jnp.float32) for _ in range(nb)],
        pltpu.SemaphoreType.DMA((nb,)),
        pltpu.SemaphoreType.DMA((nb,)),
    ]
    if with_cnt:
        out_type.append(jax.ShapeDtypeStruct((2, N, CW), jnp.float32))
        scratch += [
            pltpu.VMEM_SHARED((ACCR, CW), jnp.float32),
            pltpu.VMEM((gs // 2, K), jnp.int32),
            pltpu.VMEM((K, CW), jnp.float32),
            pltpu.SemaphoreType.DMA((nb,)),
        ]

    @functools.partial(
        pl.kernel,
        mesh=mesh,
        compiler_params=pltpu.CompilerParams(use_tc_tiling_on_sc=False),
        out_type=out_type,
        scratch_types=scratch,
    )
    def agg(*args):
        if with_cnt:
            (xstk, edges, cdst, zeros, zeros16, ones_hbm, out, outc, acc, xs,
             idx, rows, gsem, ssem, cntacc, cidx, ones_v, csem) = args
        else:
            xstk, edges, zeros, out, acc, xs, idx, rows, gsem, ssem = args
        c = lax.axis_index("c")
        s = lax.axis_index("s")
        pltpu.sync_copy(zeros.at[pl.ds(s * RPTZ, RPTZ)],
                        acc.at[pl.ds(s * RPTZ, RPTZ)])
        pltpu.sync_copy(xstk.at[pl.ds(s * RPT, RPT), pl.ds(c * HALF, HALF)],
                        xs.at[pl.ds(s * RPT, RPT)])
        if with_cnt:
            pltpu.sync_copy(zeros16.at[pl.ds(s * RPTZ, RPTZ)],
                            cntacc.at[pl.ds(s * RPTZ, RPTZ)])
            pltpu.sync_copy(ones_hbm, ones_v)
        plsc.subcore_barrier()

        nsup = NCHUNK // gs
        base = s * nsup

        def super_body(g, carry):
            pltpu.sync_copy(edges.at[pl.ds((base + g) * gs, gs)], idx)
            if with_cnt:
                pltpu.sync_copy(
                    cdst.at[c, pl.ds(s * (nsup * gs // 2) + g * (gs // 2),
                                     gs // 2)], cidx)

            def gather(j):
                b = j % nb
                return pltpu.async_copy(
                    xs.at[idx.at[j, 0]], rows[b], gsem.at[b])

            def wait_cnt(j):
                if with_cnt and j % 2 == 0:
                    cd[j].wait()

            gd = [None] * gs
            sd = [None] * gs
            cd = [None] * gs
            for j in range(nb - 1):
                gd[j] = gather(j)
            for j in range(gs):
                b = j % nb
                gd[j].wait()
                sd[j] = pltpu.async_copy(
                    rows[b], acc.at[idx.at[j, 1]], ssem.at[b], add=True)
                if with_cnt and j % 2 == 0:
                    cd[j] = pltpu.async_copy(
                        ones_v, cntacc.at[cidx.at[j // 2]],
                        csem.at[(j // 2) % nb], add=True)
                nxt = j + nb - 1
                if nxt < gs:
                    if nxt - nb >= 0:
                        sd[nxt - nb].wait()
                        wait_cnt(nxt - nb)
                    gd[nxt] = gather(nxt)
            for j in range(gs - nb, gs):
                sd[j].wait()
                wait_cnt(j)
            return carry

        lax.fori_loop(0, nsup, super_body, 0)
        plsc.subcore_barrier()
        pltpu.sync_copy(acc.at[pl.ds(s * RPT, RPT)],
                        out.at[pl.ds(s * RPT, RPT), pl.ds(c * HALF, HALF)])
        if with_cnt:
            pltpu.sync_copy(cntacc.at[pl.ds(s * RPT, RPT)],
                            outc.at[c, pl.ds(s * RPT, RPT)])

    return agg


_agg_cnt = _make_agg(True, 4, 32)
_agg = _make_agg(False, 5, 32)


BN = 2000


def _tc1_body(p, ct, xr, wl, wr, br, o):
    cnt = jnp.maximum(ct[0, :, :1] + ct[1, :, :1], 1.0)
    mean = p[...] / cnt
    z = (jnp.dot(mean, wl[...], preferred_element_type=jnp.float32)
         + jnp.dot(xr[...], wr[...], preferred_element_type=jnp.float32)
         + br[...])
    o[...] = jnp.maximum(z, 0.0)


def _tc1(sums, cnt, x, Wl1, Wr1, b1):
    return pl.pallas_call(
        _tc1_body,
        grid=(N // BN,),
        in_specs=[
            pl.BlockSpec((BN, D), lambda i: (i, 0)),
            pl.BlockSpec((2, BN, CW), lambda i: (0, i, 0)),
            pl.BlockSpec((BN, D), lambda i: (i, 0)),
            pl.BlockSpec((D, D), lambda i: (0, 0)),
            pl.BlockSpec((D, D), lambda i: (0, 0)),
            pl.BlockSpec((1, D), lambda i: (0, 0)),
        ],
        out_specs=pl.BlockSpec((BN, D), lambda i: (i, 0)),
        out_shape=jax.ShapeDtypeStruct((ACCR, D), jnp.float32),
    )(sums, cnt, x, Wl1, Wr1, b1)


def _tc2_body(a, ct, h2, wl, wr, br, o):
    cnt = jnp.maximum(ct[0, :, :1] + ct[1, :, :1], 1.0)
    mean = a[...] / cnt
    z = (jnp.dot(mean, wl[...], preferred_element_type=jnp.float32)
         + jnp.dot(h2[...], wr[...], preferred_element_type=jnp.float32)
         + br[...])
    o[...] = z


def _tc2(agg2, cnt, h2, Wl2, Wr2, b2):
    return pl.pallas_call(
        _tc2_body,
        grid=(N // BN,),
        in_specs=[
            pl.BlockSpec((BN, D), lambda i: (i, 0)),
            pl.BlockSpec((2, BN, CW), lambda i: (0, i, 0)),
            pl.BlockSpec((BN, D), lambda i: (i, 0)),
            pl.BlockSpec((D, D), lambda i: (0, 0)),
            pl.BlockSpec((D, D), lambda i: (0, 0)),
            pl.BlockSpec((1, D), lambda i: (0, 0)),
        ],
        out_specs=pl.BlockSpec((BN, D), lambda i: (i, 0)),
        out_shape=jax.ShapeDtypeStruct((N, D), jnp.float32),
    )(agg2, cnt, h2, Wl2, Wr2, b2)


def kernel(x, edge_index, Wl1, Wr1, b1, Wl2, Wr2, b2):
    src = edge_index[0].astype(jnp.int32)
    dst = edge_index[1].astype(jnp.int32)
    src_p = jnp.concatenate(
        [src, jnp.zeros((EP - E,), jnp.int32)]).reshape(NCHUNK_TOT, K)
    dst_p = jnp.concatenate(
        [dst, jnp.full((EP - E,), N, jnp.int32)]).reshape(NCHUNK_TOT, K)
    edges = jnp.stack([src_p, dst_p], axis=1)
    cdst = dst_p.reshape(2, NCHUNK_TOT // 2, K)
    zeros = jnp.zeros((ACCR, HALF), jnp.float32)
    zeros16 = jnp.zeros((ACCR, CW), jnp.float32)
    ones16 = jnp.ones((K, CW), jnp.float32)

    sums, cnt = _agg_cnt(x, edges, cdst, zeros, zeros16, ones16)
    h2 = _tc1(sums, cnt, x, Wl1, Wr1, b1.reshape(1, D))
    (agg2,) = _agg(h2, edges, zeros)
    return _tc2(agg2, cnt, h2, Wl2, Wr2, b2.reshape(1, D))
